# baseline (speedup 1.0000x reference)
import functools

import jax
import jax.numpy as jnp
from jax.experimental import pallas as pl
from jax.experimental.pallas import tpu as pltpu


def _leaky(x):
    return jnp.where(x >= 0, x, 0.01 * x)


def _gru_body(T, F, H, B, x_ref, wih_ref, whh_ref, bih_ref, bhh_ref, w1_ref,
              o_ref):
    x = x_ref[0]
    wih = wih_ref[...]
    whh = whh_ref[...]
    brz = bih_ref[0]
    bin_ = bhh_ref[0][:H]
    bhn = bhh_ref[0][H:]
    h = None
    for t in range(T):
        xt = x[:, t * F:(t + 1) * F]
        if h is None:
            gi = jnp.dot(xt, wih, preferred_element_type=jnp.float32)
            srz = gi[:, :2 * H] + brz
            q = jnp.broadcast_to(bhn, (xt.shape[0], H))
            an = gi[:, 2 * H:] + bin_
        else:
            hb = h.astype(jnp.bfloat16)
            srz = (jnp.dot(xt, wih[:, :2 * H], preferred_element_type=jnp.float32)
                   + jnp.dot(hb, whh[:, :2 * H], preferred_element_type=jnp.float32)
                   + brz)
            q = jnp.dot(hb, whh[:, 2 * H:], preferred_element_type=jnp.float32) + bhn
            an = jnp.dot(xt, wih[:, 2 * H:], preferred_element_type=jnp.float32) + bin_
        tr = jnp.tanh(srz[:, :H])
        tz = jnp.tanh(srz[:, H:])
        n = jnp.tanh(an + 0.5 * (tr * q + q))
        if h is None:
            h = (0.5 - 0.5 * tz) * n
        else:
            h = 0.5 * ((n + h) + tz * (h - n))
    y1 = jnp.dot(h.astype(jnp.bfloat16), w1_ref[...],
                 preferred_element_type=jnp.float32)
    o_ref[...] = y1.astype(jnp.bfloat16)


def _gcn1_body(B, H, a_ref, y_ref, b_ref, w2_ref, o_ref):
    u = jnp.dot(a_ref[...], y_ref[...], preferred_element_type=jnp.float32)
    t2 = _leaky(u + b_ref[0])
    w2 = w2_ref[...]
    for b in range(B):
        yb = jnp.dot(t2[:, b * H:(b + 1) * H].astype(jnp.bfloat16), w2,
                     preferred_element_type=jnp.float32)
        o_ref[:, b * H:(b + 1) * H] = yb.astype(jnp.bfloat16)


def _gcn2_body(B, H, a_ref, y_ref, b_ref, wlin_ref, blin_ref, o_ref):
    v = jnp.dot(a_ref[...], y_ref[...], preferred_element_type=jnp.float32)
    t3 = _leaky(v + b_ref[0])
    wlin = wlin_ref[...]
    blin = blin_ref[0]
    for b in range(B):
        ob = jnp.dot(t3[:, b * H:(b + 1) * H].astype(jnp.bfloat16), wlin,
                     preferred_element_type=jnp.float32) + blin
        o_ref[b] = ob


def kernel(A, X, gru_Wih, gru_Whh, gru_bih, gru_bhh, W1, b1, W2, b2, Wlin, blin):
    B, N, T, F = X.shape
    H = gru_Whh.shape[1]
    T_OUT = Wlin.shape[1]

    Xr = X.reshape(B, N, T * F).astype(jnp.bfloat16)
    Abf = A.astype(jnp.bfloat16)
    sc = jnp.concatenate([jnp.full((2 * H,), 0.5, jnp.float32),
                          jnp.ones((H,), jnp.float32)])
    wih_t = (gru_Wih.T * sc).astype(jnp.bfloat16)
    whh_t = (gru_Whh.T * sc).astype(jnp.bfloat16)
    brz2 = (0.5 * (gru_bih[:2 * H] + gru_bhh[:2 * H])).reshape(1, -1)
    bn2 = jnp.concatenate([gru_bih[2 * H:], gru_bhh[2 * H:]]).reshape(1, -1)
    b1t = jnp.tile(b1, B).reshape(1, B * H)
    b2t = jnp.tile(b2, B).reshape(1, B * H)

    BM_G = min(N, 1024)
    BM_A = min(N, 256)

    y1 = pl.pallas_call(
        functools.partial(_gru_body, T, F, H, B),
        grid=(B, N // BM_G),
        in_specs=[
            pl.BlockSpec((1, BM_G, T * F), lambda b, j: (b, j, 0)),
            pl.BlockSpec((F, 3 * H), lambda b, j: (0, 0)),
            pl.BlockSpec((H, 3 * H), lambda b, j: (0, 0)),
            pl.BlockSpec((1, 2 * H), lambda b, j: (0, 0)),
            pl.BlockSpec((1, 2 * H), lambda b, j: (0, 0)),
            pl.BlockSpec((H, H), lambda b, j: (0, 0)),
        ],
        out_specs=pl.BlockSpec((BM_G, H), lambda b, j: (j, b)),
        out_shape=jax.ShapeDtypeStruct((N, B * H), jnp.bfloat16),
        compiler_params=pltpu.CompilerParams(
            dimension_semantics=("parallel", "parallel")),
    )(Xr, wih_t, whh_t, brz2, bn2, W1.astype(jnp.bfloat16))

    y2 = pl.pallas_call(
        functools.partial(_gcn1_body, B, H),
        grid=(N // BM_A,),
        in_specs=[
            pl.BlockSpec((BM_A, N), lambda j: (j, 0)),
            pl.BlockSpec((N, B * H), lambda j: (0, 0)),
            pl.BlockSpec((1, B * H), lambda j: (0, 0)),
            pl.BlockSpec((H, H), lambda j: (0, 0)),
        ],
        out_specs=pl.BlockSpec((BM_A, B * H), lambda j: (j, 0)),
        out_shape=jax.ShapeDtypeStruct((N, B * H), jnp.bfloat16),
        compiler_params=pltpu.CompilerParams(
            dimension_semantics=("parallel",)),
    )(Abf, y1, b1t, W2.astype(jnp.bfloat16))

    out = pl.pallas_call(
        functools.partial(_gcn2_body, B, H),
        grid=(N // BM_A,),
        in_specs=[
            pl.BlockSpec((BM_A, N), lambda j: (j, 0)),
            pl.BlockSpec((N, B * H), lambda j: (0, 0)),
            pl.BlockSpec((1, B * H), lambda j: (0, 0)),
            pl.BlockSpec((H, T_OUT), lambda j: (0, 0)),
            pl.BlockSpec((1, T_OUT), lambda j: (0, 0)),
        ],
        out_specs=pl.BlockSpec((B, BM_A, T_OUT), lambda j: (0, j, 0)),
        out_shape=jax.ShapeDtypeStruct((B, N, T_OUT), jnp.float32),
        compiler_params=pltpu.CompilerParams(
            dimension_semantics=("parallel",)),
    )(Abf, y2, b2t, Wlin.astype(jnp.bfloat16), blin.reshape(1, -1))

    return out
